# baseline (speedup 1.0000x reference)
import functools

import jax
import jax.numpy as jnp
from jax import lax
from jax.experimental import pallas as pl
from jax.experimental.pallas import tpu as pltpu
from jax.experimental.pallas import tpu_sc as plsc

HID = 128
N_PROT_ = 8000
N_LIG_ = 2000
N_NODES_ = N_PROT_ + N_LIG_
N_EDGES_ = 320000
NUM_GRAPHS_ = 16
NUM_TIMESTEPS_ = 1000

EDGE_BLK = 6400
NODE_BLK = 2000

SC_CORES = 2
SC_SUBCORES = 16
SC_WORKERS = SC_CORES * SC_SUBCORES
EDGES_PER_WORKER = N_EDGES_ // SC_WORKERS
SC_CHUNK = 400
SC_NCHUNK = EDGES_PER_WORKER // SC_CHUNK
NODES_PER_CORE = N_NODES_ // SC_CORES
ACC_ROWS = NODES_PER_CORE + 8
ROWS_PER_TILE = 312
TAIL_ROWS = ACC_ROWS - SC_SUBCORES * ROWS_PER_TILE
SCAT_CHUNKS = N_EDGES_ // SC_SUBCORES // SC_CHUNK

_sc_mesh = plsc.VectorSubcoreMesh(core_axis_name="c", subcore_axis_name="s")


POSW = 16


@functools.partial(
    pl.kernel,
    mesh=_sc_mesh,
    compiler_params=pltpu.CompilerParams(use_tc_tiling_on_sc=False),
    out_type=[
        jax.ShapeDtypeStruct((N_EDGES_, HID), jnp.bfloat16),
        jax.ShapeDtypeStruct((N_EDGES_, HID), jnp.bfloat16),
        jax.ShapeDtypeStruct((N_EDGES_, POSW), jnp.float32),
        jax.ShapeDtypeStruct((N_EDGES_, POSW), jnp.float32),
    ],
    scratch_types=[
        pltpu.VMEM((SC_CHUNK,), jnp.int32),
        pltpu.VMEM((SC_CHUNK,), jnp.int32),
        pltpu.VMEM((SC_CHUNK, HID), jnp.bfloat16),
        pltpu.VMEM((SC_CHUNK, HID), jnp.bfloat16),
        pltpu.VMEM((SC_CHUNK, POSW), jnp.float32),
        pltpu.VMEM((SC_CHUNK, POSW), jnp.float32),
        pltpu.SemaphoreType.DMA,
        pltpu.SemaphoreType.DMA,
    ],
)
def _sc_gather(td_hbm, ts_hbm, posw_hbm, dst_hbm, src_hbm,
               gd_hbm, gs_hbm, pd_hbm, ps_hbm,
               idxd_v, idxs_v, bufd, bufs, bpd, bps, semd, sems):
    wid = lax.axis_index("s") * SC_CORES + lax.axis_index("c")

    @pl.loop(0, SC_NCHUNK)
    def _(i):
        base = wid * EDGES_PER_WORKER + i * SC_CHUNK
        pltpu.sync_copy(dst_hbm.at[pl.ds(base, SC_CHUNK)], idxd_v)
        pltpu.sync_copy(src_hbm.at[pl.ds(base, SC_CHUNK)], idxs_v)
        cpd = pltpu.async_copy(td_hbm.at[idxd_v], bufd, semd)
        cps = pltpu.async_copy(ts_hbm.at[idxs_v], bufs, sems)
        cpp = pltpu.async_copy(posw_hbm.at[idxd_v], bpd, semd)
        cpq = pltpu.async_copy(posw_hbm.at[idxs_v], bps, sems)
        cpd.wait()
        cps.wait()
        cpp.wait()
        cpq.wait()
        pltpu.sync_copy(bufd, gd_hbm.at[pl.ds(base, SC_CHUNK)])
        pltpu.sync_copy(bufs, gs_hbm.at[pl.ds(base, SC_CHUNK)])
        pltpu.sync_copy(bpd, pd_hbm.at[pl.ds(base, SC_CHUNK)])
        pltpu.sync_copy(bps, ps_hbm.at[pl.ds(base, SC_CHUNK)])


@functools.partial(
    pl.kernel,
    mesh=_sc_mesh,
    compiler_params=pltpu.CompilerParams(use_tc_tiling_on_sc=False),
    out_type=[
        jax.ShapeDtypeStruct((SC_CORES, ACC_ROWS, HID), jnp.float32),
        jax.ShapeDtypeStruct((SC_CORES, ACC_ROWS, POSW), jnp.float32),
    ],
    scratch_types=[
        pltpu.VMEM((SC_CHUNK,), jnp.int32),
        pltpu.VMEM((SC_CHUNK, HID), jnp.float32),
        pltpu.VMEM((SC_CHUNK, POSW), jnp.float32),
        pltpu.VMEM_SHARED((ACC_ROWS, HID), jnp.float32),
        pltpu.VMEM_SHARED((ACC_ROWS, POSW), jnp.float32),
    ],
)
def _sc_scatter(mt_hbm, tr_hbm, dst_hbm, zeros_hbm, zeros16_hbm,
                out_hbm, outx_hbm, idx_v, buf, bufx, accum, accx):
    cid = lax.axis_index("c")
    sid = lax.axis_index("s")
    row0 = sid * ROWS_PER_TILE
    pltpu.sync_copy(zeros_hbm, buf)
    pltpu.sync_copy(zeros16_hbm, bufx)
    pltpu.sync_copy(buf.at[pl.ds(0, ROWS_PER_TILE)],
                    accum.at[pl.ds(row0, ROWS_PER_TILE)])
    pltpu.sync_copy(bufx.at[pl.ds(0, ROWS_PER_TILE)],
                    accx.at[pl.ds(row0, ROWS_PER_TILE)])

    @pl.when(sid == SC_SUBCORES - 1)
    def _():
        t0 = SC_SUBCORES * ROWS_PER_TILE
        pltpu.sync_copy(buf.at[pl.ds(0, TAIL_ROWS)],
                        accum.at[pl.ds(t0, TAIL_ROWS)])
        pltpu.sync_copy(bufx.at[pl.ds(0, TAIL_ROWS)],
                        accx.at[pl.ds(t0, TAIL_ROWS)])

    plsc.subcore_barrier()
    nbase = cid * NODES_PER_CORE

    @pl.loop(0, SCAT_CHUNKS)
    def _(i):
        base = sid * (N_EDGES_ // SC_SUBCORES) + i * SC_CHUNK
        pltpu.sync_copy(dst_hbm.at[pl.ds(base, SC_CHUNK)], idx_v)
        pltpu.sync_copy(mt_hbm.at[pl.ds(base, SC_CHUNK)], buf)
        pltpu.sync_copy(tr_hbm.at[pl.ds(base, SC_CHUNK)], bufx)

        @pl.loop(0, SC_CHUNK // 16)
        def _(j):
            idx16 = idx_v[pl.ds(j * 16, 16)] - nbase
            ok = (idx16 >= 0) & (idx16 < NODES_PER_CORE)
            idx_v[pl.ds(j * 16, 16)] = jnp.where(
                ok, idx16, jnp.full((16,), NODES_PER_CORE, jnp.int32))

        pltpu.sync_copy(buf, accum.at[idx_v], add=True)
        pltpu.sync_copy(bufx, accx.at[idx_v], add=True)

    plsc.subcore_barrier()

    @pl.loop(0, ROWS_PER_TILE // 104)
    def _(k):
        r = row0 + k * 104
        pltpu.sync_copy(accum.at[pl.ds(r, 104)], buf.at[pl.ds(0, 104)])
        pltpu.sync_copy(buf.at[pl.ds(0, 104)],
                        out_hbm.at[cid, pl.ds(r, 104)])
        pltpu.sync_copy(accx.at[pl.ds(r, 104)], bufx.at[pl.ds(0, 104)])
        pltpu.sync_copy(bufx.at[pl.ds(0, 104)],
                        outx_hbm.at[cid, pl.ds(r, 104)])

    @pl.when(sid == SC_SUBCORES - 1)
    def _():
        t0 = SC_SUBCORES * ROWS_PER_TILE
        pltpu.sync_copy(accum.at[pl.ds(t0, TAIL_ROWS)],
                        buf.at[pl.ds(0, TAIL_ROWS)])
        pltpu.sync_copy(buf.at[pl.ds(0, TAIL_ROWS)],
                        out_hbm.at[cid, pl.ds(t0, TAIL_ROWS)])
        pltpu.sync_copy(accx.at[pl.ds(t0, TAIL_ROWS)],
                        bufx.at[pl.ds(0, TAIL_ROWS)])
        pltpu.sync_copy(bufx.at[pl.ds(0, TAIL_ROWS)],
                        outx_hbm.at[cid, pl.ds(t0, TAIL_ROWS)])


def _mid_body(gd_ref, gs_ref, pd_ref, ps_ref, wd_ref, b1_ref, w2_ref, b2_ref,
              wx_ref, bx_ref, m2_ref, trans_ref):
    rel = pd_ref[...] - ps_ref[...]
    d2 = jnp.sum(rel * rel, axis=1, keepdims=True)
    z = (gd_ref[...] + gs_ref[...]
         + (d2 * wd_ref[...] + b1_ref[...]).astype(jnp.bfloat16))
    m1 = z * jax.nn.sigmoid(z)
    y = lax.dot_general(
        m1, w2_ref[...].astype(jnp.bfloat16),
        (((1,), (0,)), ((), ())), preferred_element_type=jnp.float32)
    y = y + b2_ref[...]
    m2 = y * jax.nn.sigmoid(y)
    m2_ref[...] = m2
    coef = lax.dot_general(
        m2.astype(jnp.bfloat16), wx_ref[...].astype(jnp.bfloat16),
        (((1,), (0,)), ((), ())), preferred_element_type=jnp.float32)
    coef = coef[:, 0:1] + bx_ref[0, 0]
    trans_ref[...] = rel * (coef / (jnp.sqrt(d2) + 1.0))


def _edge_mlp(gd, gs, pd, ps, w_d2, b1, W2, b2, Wx8, bx8):
    full = lambda i: (0, 0)
    return pl.pallas_call(
        _mid_body,
        grid=(N_EDGES_ // EDGE_BLK,),
        in_specs=[
            pl.BlockSpec((EDGE_BLK, HID), lambda i: (i, 0)),
            pl.BlockSpec((EDGE_BLK, HID), lambda i: (i, 0)),
            pl.BlockSpec((EDGE_BLK, POSW), lambda i: (i, 0)),
            pl.BlockSpec((EDGE_BLK, POSW), lambda i: (i, 0)),
            pl.BlockSpec((1, HID), full),
            pl.BlockSpec((1, HID), full),
            pl.BlockSpec((HID, HID), full),
            pl.BlockSpec((1, HID), full),
            pl.BlockSpec((HID, 8), full),
            pl.BlockSpec((1, 8), full),
        ],
        out_specs=[
            pl.BlockSpec((EDGE_BLK, HID), lambda i: (i, 0)),
            pl.BlockSpec((EDGE_BLK, POSW), lambda i: (i, 0)),
        ],
        out_shape=[
            jax.ShapeDtypeStruct((N_EDGES_, HID), jnp.float32),
            jax.ShapeDtypeStruct((N_EDGES_, POSW), jnp.float32),
        ],
    )(gd, gs, pd, ps, w_d2, b1, W2, b2, Wx8, bx8)


def _node_body(h_ref, agg_ref, wh1_ref, wh2_ref, bh_ref, h2_ref):
    h = h_ref[...]
    agg = agg_ref[...]
    u = lax.dot_general(
        h.astype(jnp.bfloat16), wh1_ref[...].astype(jnp.bfloat16),
        (((1,), (0,)), ((), ())), preferred_element_type=jnp.float32)
    u = u + lax.dot_general(
        agg.astype(jnp.bfloat16), wh2_ref[...].astype(jnp.bfloat16),
        (((1,), (0,)), ((), ())), preferred_element_type=jnp.float32)
    u = u + bh_ref[...]
    h2_ref[...] = h + u * jax.nn.sigmoid(u)


def _node_update(h, agg, Wh1, Wh2, bh):
    full = lambda i: (0, 0)
    return pl.pallas_call(
        _node_body,
        grid=(N_NODES_ // NODE_BLK,),
        in_specs=[
            pl.BlockSpec((NODE_BLK, HID), lambda i: (i, 0)),
            pl.BlockSpec((NODE_BLK, HID), lambda i: (i, 0)),
            pl.BlockSpec((HID, HID), full),
            pl.BlockSpec((HID, HID), full),
            pl.BlockSpec((1, HID), full),
        ],
        out_specs=pl.BlockSpec((NODE_BLK, HID), lambda i: (i, 0)),
        out_shape=jax.ShapeDtypeStruct((N_NODES_, HID), jnp.float32),
    )(h, agg, Wh1, Wh2, bh)


def kernel(protein_pos, protein_v, init_ligand_pos, W_prot, b_prot, W_lig,
           b_lig, W_e1, b_e1, W_e2, b_e2, W_h, b_h, W_x, b_x, W_v, b_v,
           batch_protein, init_ligand_v, batch_ligand, time_step, edge_index):
    sums = jax.ops.segment_sum(protein_pos, batch_protein,
                               num_segments=NUM_GRAPHS_)
    cnt = jax.ops.segment_sum(jnp.ones((N_PROT_,), jnp.float32),
                              batch_protein, num_segments=NUM_GRAPHS_)
    offset = sums / jnp.maximum(cnt, 1.0)[:, None]
    off_lig = offset[batch_ligand]
    p_pos = protein_pos - offset[batch_protein]
    l_pos = init_ligand_pos - off_lig
    lig_onehot = jax.nn.one_hot(init_ligand_v, 13, dtype=jnp.float32)
    t_feat = (time_step.astype(jnp.float32) / NUM_TIMESTEPS_)[batch_ligand][:, None]
    lig_feat = jnp.concatenate([lig_onehot, t_feat], axis=-1)
    h_prot = protein_v @ W_prot + b_prot
    h_lig = lig_feat @ W_lig + b_lig
    h_prot = jnp.concatenate([h_prot, jnp.zeros((N_PROT_, 1), jnp.float32)], axis=-1)
    h_lig = jnp.concatenate([h_lig, jnp.ones((N_LIG_, 1), jnp.float32)], axis=-1)
    h = jnp.concatenate([h_prot, h_lig], axis=0)
    pos = jnp.concatenate([p_pos, l_pos], axis=0)
    W1d = W_e1[:HID]
    W1s = W_e1[HID:2 * HID]
    w_d2 = W_e1[2 * HID:2 * HID + 1]
    Td = (h @ W1d).astype(jnp.bfloat16)
    Ts = (h @ W1s).astype(jnp.bfloat16)
    src = edge_index[0]
    dst = edge_index[1]
    posw = jnp.pad(pos, ((0, 0), (0, POSW - 3)))
    gd, gs, pd, ps = _sc_gather(Td, Ts, posw, dst, src)
    Wx8 = jnp.pad(W_x, ((0, 0), (0, 7)))
    m2, trans = _edge_mlp(gd, gs, pd, ps, w_d2, b_e1[None, :], W_e2,
                          b_e2[None, :], Wx8, jnp.pad(b_x, (0, 7))[None, :])
    zeros_acc = jnp.zeros((SC_CHUNK, HID), jnp.float32)
    zeros16 = jnp.zeros((SC_CHUNK, POSW), jnp.float32)
    parts, partsx = _sc_scatter(m2, trans, dst, zeros_acc, zeros16)
    agg = jnp.concatenate([parts[0, :NODES_PER_CORE],
                           parts[1, :NODES_PER_CORE]], axis=0)
    x_agg = jnp.concatenate([partsx[0, :NODES_PER_CORE],
                             partsx[1, :NODES_PER_CORE]], axis=0)[:, :3]
    h2 = _node_update(h, agg, W_h[:HID], W_h[HID:], b_h[None, :])
    mask_ligand = jnp.concatenate([jnp.zeros((N_PROT_,), jnp.float32),
                                   jnp.ones((N_LIG_,), jnp.float32)])[:, None]
    pos2 = pos + x_agg * mask_ligand
    pred_ligand_pos = pos2[N_PROT_:] + off_lig
    pred_ligand_v = h2[N_PROT_:] @ W_v + b_v
    return pred_ligand_pos, pred_ligand_v


# speedup vs baseline: 1.5072x; 1.5072x over previous
import functools

import jax
import jax.numpy as jnp
from jax import lax
from jax.experimental import pallas as pl
from jax.experimental.pallas import tpu as pltpu
from jax.experimental.pallas import tpu_sc as plsc

HID = 128
N_PROT_ = 8000
N_LIG_ = 2000
N_NODES_ = N_PROT_ + N_LIG_
N_EDGES_ = 320000
NUM_GRAPHS_ = 16
NUM_TIMESTEPS_ = 1000

EDGE_BLK = 6400
NODE_BLK = 2000

SC_CORES = 2
SC_SUBCORES = 16
SC_WORKERS = SC_CORES * SC_SUBCORES
K_SLICES = 2
E_SLICE = N_EDGES_ // K_SLICES
EDGES_PER_WORKER = E_SLICE // SC_WORKERS
G_CHUNK = 200
SC_NCHUNK = EDGES_PER_WORKER // G_CHUNK
SC_CHUNK = 400
NODES_PER_CORE = N_NODES_ // SC_CORES
ACC_ROWS = NODES_PER_CORE + 8
ROWS_PER_TILE = 312
TAIL_ROWS = ACC_ROWS - SC_SUBCORES * ROWS_PER_TILE
SCAT_CHUNKS = E_SLICE // SC_SUBCORES // SC_CHUNK

_sc_mesh = plsc.VectorSubcoreMesh(core_axis_name="c", subcore_axis_name="s")


POSW = 16


@functools.partial(
    pl.kernel,
    mesh=_sc_mesh,
    compiler_params=pltpu.CompilerParams(use_tc_tiling_on_sc=False),
    out_type=[
        jax.ShapeDtypeStruct((E_SLICE, HID), jnp.float32),
        jax.ShapeDtypeStruct((E_SLICE, HID), jnp.float32),
        jax.ShapeDtypeStruct((E_SLICE, POSW), jnp.float32),
        jax.ShapeDtypeStruct((E_SLICE, POSW), jnp.float32),
    ],
    scratch_types=[
        pltpu.VMEM((G_CHUNK,), jnp.int32),
        pltpu.VMEM((G_CHUNK,), jnp.int32),
        pltpu.VMEM((G_CHUNK, HID), jnp.float32),
        pltpu.VMEM((G_CHUNK, HID), jnp.float32),
        pltpu.VMEM((G_CHUNK, POSW), jnp.float32),
        pltpu.VMEM((G_CHUNK, POSW), jnp.float32),
        pltpu.SemaphoreType.DMA,
        pltpu.SemaphoreType.DMA,
    ],
)
def _sc_gather(td_hbm, ts_hbm, posw_hbm, dst_hbm, src_hbm,
               gd_hbm, gs_hbm, pd_hbm, ps_hbm,
               idxd_v, idxs_v, bufd, bufs, bpd, bps, semd, sems):
    wid = lax.axis_index("s") * SC_CORES + lax.axis_index("c")

    @pl.loop(0, SC_NCHUNK)
    def _(i):
        base = wid * EDGES_PER_WORKER + i * G_CHUNK
        pltpu.sync_copy(dst_hbm.at[pl.ds(base, G_CHUNK)], idxd_v)
        pltpu.sync_copy(src_hbm.at[pl.ds(base, G_CHUNK)], idxs_v)
        cpd = pltpu.async_copy(td_hbm.at[idxd_v], bufd, semd)
        cps = pltpu.async_copy(ts_hbm.at[idxs_v], bufs, sems)
        cpp = pltpu.async_copy(posw_hbm.at[idxd_v], bpd, semd)
        cpq = pltpu.async_copy(posw_hbm.at[idxs_v], bps, sems)
        cpd.wait()
        cps.wait()
        cpp.wait()
        cpq.wait()
        pltpu.sync_copy(bufd, gd_hbm.at[pl.ds(base, G_CHUNK)])
        pltpu.sync_copy(bufs, gs_hbm.at[pl.ds(base, G_CHUNK)])
        pltpu.sync_copy(bpd, pd_hbm.at[pl.ds(base, G_CHUNK)])
        pltpu.sync_copy(bps, ps_hbm.at[pl.ds(base, G_CHUNK)])


@functools.partial(
    pl.kernel,
    mesh=_sc_mesh,
    compiler_params=pltpu.CompilerParams(use_tc_tiling_on_sc=False),
    out_type=[
        jax.ShapeDtypeStruct((SC_CORES, ACC_ROWS, HID), jnp.float32),
        jax.ShapeDtypeStruct((SC_CORES, ACC_ROWS, POSW), jnp.float32),
    ],
    scratch_types=[
        pltpu.VMEM((SC_CHUNK,), jnp.int32),
        pltpu.VMEM((SC_CHUNK, HID), jnp.float32),
        pltpu.VMEM((SC_CHUNK, POSW), jnp.float32),
        pltpu.VMEM_SHARED((ACC_ROWS, HID), jnp.float32),
        pltpu.VMEM_SHARED((ACC_ROWS, POSW), jnp.float32),
    ],
)
def _sc_scatter(mt_hbm, tr_hbm, dst_hbm, zeros_hbm, zeros16_hbm,
                out_hbm, outx_hbm, idx_v, buf, bufx, accum, accx):
    cid = lax.axis_index("c")
    sid = lax.axis_index("s")
    row0 = sid * ROWS_PER_TILE
    pltpu.sync_copy(zeros_hbm, buf)
    pltpu.sync_copy(zeros16_hbm, bufx)
    pltpu.sync_copy(buf.at[pl.ds(0, ROWS_PER_TILE)],
                    accum.at[pl.ds(row0, ROWS_PER_TILE)])
    pltpu.sync_copy(bufx.at[pl.ds(0, ROWS_PER_TILE)],
                    accx.at[pl.ds(row0, ROWS_PER_TILE)])

    @pl.when(sid == SC_SUBCORES - 1)
    def _():
        t0 = SC_SUBCORES * ROWS_PER_TILE
        pltpu.sync_copy(buf.at[pl.ds(0, TAIL_ROWS)],
                        accum.at[pl.ds(t0, TAIL_ROWS)])
        pltpu.sync_copy(bufx.at[pl.ds(0, TAIL_ROWS)],
                        accx.at[pl.ds(t0, TAIL_ROWS)])

    plsc.subcore_barrier()
    nbase = cid * NODES_PER_CORE

    @pl.loop(0, SCAT_CHUNKS)
    def _(i):
        base = sid * (E_SLICE // SC_SUBCORES) + i * SC_CHUNK
        pltpu.sync_copy(dst_hbm.at[pl.ds(base, SC_CHUNK)], idx_v)
        pltpu.sync_copy(mt_hbm.at[pl.ds(base, SC_CHUNK)], buf)
        pltpu.sync_copy(tr_hbm.at[pl.ds(base, SC_CHUNK)], bufx)

        @pl.loop(0, SC_CHUNK // 16)
        def _(j):
            idx16 = idx_v[pl.ds(j * 16, 16)] - nbase
            ok = (idx16 >= 0) & (idx16 < NODES_PER_CORE)
            idx_v[pl.ds(j * 16, 16)] = jnp.where(
                ok, idx16, jnp.full((16,), NODES_PER_CORE, jnp.int32))

        pltpu.sync_copy(buf, accum.at[idx_v], add=True)
        pltpu.sync_copy(bufx, accx.at[idx_v], add=True)

    plsc.subcore_barrier()

    @pl.loop(0, ROWS_PER_TILE // 104)
    def _(k):
        r = row0 + k * 104
        pltpu.sync_copy(accum.at[pl.ds(r, 104)], buf.at[pl.ds(0, 104)])
        pltpu.sync_copy(buf.at[pl.ds(0, 104)],
                        out_hbm.at[cid, pl.ds(r, 104)])
        pltpu.sync_copy(accx.at[pl.ds(r, 104)], bufx.at[pl.ds(0, 104)])
        pltpu.sync_copy(bufx.at[pl.ds(0, 104)],
                        outx_hbm.at[cid, pl.ds(r, 104)])

    @pl.when(sid == SC_SUBCORES - 1)
    def _():
        t0 = SC_SUBCORES * ROWS_PER_TILE
        pltpu.sync_copy(accum.at[pl.ds(t0, TAIL_ROWS)],
                        buf.at[pl.ds(0, TAIL_ROWS)])
        pltpu.sync_copy(buf.at[pl.ds(0, TAIL_ROWS)],
                        out_hbm.at[cid, pl.ds(t0, TAIL_ROWS)])
        pltpu.sync_copy(accx.at[pl.ds(t0, TAIL_ROWS)],
                        bufx.at[pl.ds(0, TAIL_ROWS)])
        pltpu.sync_copy(bufx.at[pl.ds(0, TAIL_ROWS)],
                        outx_hbm.at[cid, pl.ds(t0, TAIL_ROWS)])


def _mid_body(gd_ref, gs_ref, pd_ref, ps_ref, wd_ref, b1_ref, w2_ref, b2_ref,
              wx_ref, bx_ref, m2_ref, trans_ref):
    rel = pd_ref[...] - ps_ref[...]
    d2 = jnp.sum(rel * rel, axis=1, keepdims=True)
    z = gd_ref[...] + gs_ref[...] + d2 * wd_ref[...] + b1_ref[...]
    m1 = z * jax.nn.sigmoid(z)
    y = lax.dot_general(
        m1.astype(jnp.bfloat16), w2_ref[...].astype(jnp.bfloat16),
        (((1,), (0,)), ((), ())), preferred_element_type=jnp.float32)
    y = y + b2_ref[...]
    m2 = y * jax.nn.sigmoid(y)
    m2_ref[...] = m2
    coef = lax.dot_general(
        m2.astype(jnp.bfloat16), wx_ref[...].astype(jnp.bfloat16),
        (((1,), (0,)), ((), ())), preferred_element_type=jnp.float32)
    coef = coef[:, 0:1] + bx_ref[0, 0]
    trans_ref[...] = rel * (coef / (jnp.sqrt(d2) + 1.0))


def _edge_mlp(gd, gs, pd, ps, w_d2, b1, W2, b2, Wx8, bx8):
    full = lambda i: (0, 0)
    return pl.pallas_call(
        _mid_body,
        grid=(E_SLICE // EDGE_BLK,),
        in_specs=[
            pl.BlockSpec((EDGE_BLK, HID), lambda i: (i, 0)),
            pl.BlockSpec((EDGE_BLK, HID), lambda i: (i, 0)),
            pl.BlockSpec((EDGE_BLK, POSW), lambda i: (i, 0)),
            pl.BlockSpec((EDGE_BLK, POSW), lambda i: (i, 0)),
            pl.BlockSpec((1, HID), full),
            pl.BlockSpec((1, HID), full),
            pl.BlockSpec((HID, HID), full),
            pl.BlockSpec((1, HID), full),
            pl.BlockSpec((HID, 8), full),
            pl.BlockSpec((1, 8), full),
        ],
        out_specs=[
            pl.BlockSpec((EDGE_BLK, HID), lambda i: (i, 0)),
            pl.BlockSpec((EDGE_BLK, POSW), lambda i: (i, 0)),
        ],
        out_shape=[
            jax.ShapeDtypeStruct((E_SLICE, HID), jnp.float32),
            jax.ShapeDtypeStruct((E_SLICE, POSW), jnp.float32),
        ],
    )(gd, gs, pd, ps, w_d2, b1, W2, b2, Wx8, bx8)


def _node_body(h_ref, agg_ref, wh1_ref, wh2_ref, bh_ref, h2_ref):
    h = h_ref[...]
    agg = agg_ref[...]
    u = lax.dot_general(
        h.astype(jnp.bfloat16), wh1_ref[...].astype(jnp.bfloat16),
        (((1,), (0,)), ((), ())), preferred_element_type=jnp.float32)
    u = u + lax.dot_general(
        agg.astype(jnp.bfloat16), wh2_ref[...].astype(jnp.bfloat16),
        (((1,), (0,)), ((), ())), preferred_element_type=jnp.float32)
    u = u + bh_ref[...]
    h2_ref[...] = h + u * jax.nn.sigmoid(u)


def _node_update(h, agg, Wh1, Wh2, bh):
    full = lambda i: (0, 0)
    return pl.pallas_call(
        _node_body,
        grid=(N_NODES_ // NODE_BLK,),
        in_specs=[
            pl.BlockSpec((NODE_BLK, HID), lambda i: (i, 0)),
            pl.BlockSpec((NODE_BLK, HID), lambda i: (i, 0)),
            pl.BlockSpec((HID, HID), full),
            pl.BlockSpec((HID, HID), full),
            pl.BlockSpec((1, HID), full),
        ],
        out_specs=pl.BlockSpec((NODE_BLK, HID), lambda i: (i, 0)),
        out_shape=jax.ShapeDtypeStruct((N_NODES_, HID), jnp.float32),
    )(h, agg, Wh1, Wh2, bh)


def kernel(protein_pos, protein_v, init_ligand_pos, W_prot, b_prot, W_lig,
           b_lig, W_e1, b_e1, W_e2, b_e2, W_h, b_h, W_x, b_x, W_v, b_v,
           batch_protein, init_ligand_v, batch_ligand, time_step, edge_index):
    sums = jax.ops.segment_sum(protein_pos, batch_protein,
                               num_segments=NUM_GRAPHS_)
    cnt = jax.ops.segment_sum(jnp.ones((N_PROT_,), jnp.float32),
                              batch_protein, num_segments=NUM_GRAPHS_)
    offset = sums / jnp.maximum(cnt, 1.0)[:, None]
    off_lig = offset[batch_ligand]
    p_pos = protein_pos - offset[batch_protein]
    l_pos = init_ligand_pos - off_lig
    lig_onehot = jax.nn.one_hot(init_ligand_v, 13, dtype=jnp.float32)
    t_feat = (time_step.astype(jnp.float32) / NUM_TIMESTEPS_)[batch_ligand][:, None]
    lig_feat = jnp.concatenate([lig_onehot, t_feat], axis=-1)
    h_prot = protein_v @ W_prot + b_prot
    h_lig = lig_feat @ W_lig + b_lig
    h_prot = jnp.concatenate([h_prot, jnp.zeros((N_PROT_, 1), jnp.float32)], axis=-1)
    h_lig = jnp.concatenate([h_lig, jnp.ones((N_LIG_, 1), jnp.float32)], axis=-1)
    h = jnp.concatenate([h_prot, h_lig], axis=0)
    pos = jnp.concatenate([p_pos, l_pos], axis=0)
    W1d = W_e1[:HID]
    W1s = W_e1[HID:2 * HID]
    w_d2 = W_e1[2 * HID:2 * HID + 1]
    Td = h @ W1d
    Ts = h @ W1s
    src = edge_index[0]
    dst = edge_index[1]
    posw = jnp.pad(pos, ((0, 0), (0, POSW - 3)))
    Wx8 = jnp.pad(W_x, ((0, 0), (0, 7)))
    bx8 = jnp.pad(b_x, (0, 7))[None, :]
    zeros_acc = jnp.zeros((SC_CHUNK, HID), jnp.float32)
    zeros16 = jnp.zeros((SC_CHUNK, POSW), jnp.float32)
    gath = []
    for s in range(K_SLICES):
        sl = slice(s * E_SLICE, (s + 1) * E_SLICE)
        gath.append((_sc_gather(Td, Ts, posw, dst[sl], src[sl]), dst[sl]))
    mids = []
    for (gd, gs, pd, ps), dsl in gath:
        m2, trans = _edge_mlp(gd, gs, pd, ps, w_d2, b_e1[None, :], W_e2,
                              b_e2[None, :], Wx8, bx8)
        mids.append((m2, trans, dsl))
    agg_p = []
    x_p = []
    for m2, trans, dsl in mids:
        parts, partsx = _sc_scatter(m2, trans, dsl, zeros_acc, zeros16)
        agg_p.append(parts)
        x_p.append(partsx)
    parts = agg_p[0] + agg_p[1]
    partsx = x_p[0] + x_p[1]
    agg = jnp.concatenate([parts[0, :NODES_PER_CORE],
                           parts[1, :NODES_PER_CORE]], axis=0)
    x_agg = jnp.concatenate([partsx[0, :NODES_PER_CORE],
                             partsx[1, :NODES_PER_CORE]], axis=0)[:, :3]
    h2 = _node_update(h, agg, W_h[:HID], W_h[HID:], b_h[None, :])
    mask_ligand = jnp.concatenate([jnp.zeros((N_PROT_,), jnp.float32),
                                   jnp.ones((N_LIG_,), jnp.float32)])[:, None]
    pos2 = pos + x_agg * mask_ligand
    pred_ligand_pos = pos2[N_PROT_:] + off_lig
    pred_ligand_v = h2[N_PROT_:] @ W_v + b_v
    return pred_ligand_pos, pred_ligand_v


# speedup vs baseline: 1.7225x; 1.1428x over previous
import functools

import jax
import jax.numpy as jnp
from jax import lax
from jax.experimental import pallas as pl
from jax.experimental.pallas import tpu as pltpu
from jax.experimental.pallas import tpu_sc as plsc

HID = 128
N_PROT_ = 8000
N_LIG_ = 2000
N_NODES_ = N_PROT_ + N_LIG_
N_EDGES_ = 320000
NUM_GRAPHS_ = 16
NUM_TIMESTEPS_ = 1000

EDGE_BLK = 6400
NODE_BLK = 2000

SC_CORES = 2
SC_SUBCORES = 16
SC_WORKERS = SC_CORES * SC_SUBCORES
K_SLICES = 2
E_SLICE = N_EDGES_ // K_SLICES
EDGES_PER_WORKER = E_SLICE // SC_WORKERS
G_CHUNK = 200
SC_NCHUNK = EDGES_PER_WORKER // G_CHUNK
SC_CHUNK = 400
LIG_N = N_LIG_
ACC_ROWS = 2048
ROWS_PER_TILE = ACC_ROWS // SC_SUBCORES
S_CHUNK = 200
EDGES_PER_CORE = E_SLICE // SC_CORES
SCAT_CHUNKS = EDGES_PER_CORE // SC_SUBCORES // S_CHUNK

_sc_mesh = plsc.VectorSubcoreMesh(core_axis_name="c", subcore_axis_name="s")


POSW = 16


@functools.partial(
    pl.kernel,
    mesh=_sc_mesh,
    compiler_params=pltpu.CompilerParams(use_tc_tiling_on_sc=False),
    out_type=[
        jax.ShapeDtypeStruct((E_SLICE, HID), jnp.float32),
        jax.ShapeDtypeStruct((E_SLICE, HID), jnp.float32),
        jax.ShapeDtypeStruct((E_SLICE, POSW), jnp.float32),
        jax.ShapeDtypeStruct((E_SLICE, POSW), jnp.float32),
    ],
    scratch_types=[
        pltpu.VMEM((G_CHUNK,), jnp.int32),
        pltpu.VMEM((G_CHUNK,), jnp.int32),
        pltpu.VMEM((G_CHUNK, HID), jnp.float32),
        pltpu.VMEM((G_CHUNK, HID), jnp.float32),
        pltpu.VMEM((G_CHUNK, POSW), jnp.float32),
        pltpu.VMEM((G_CHUNK, POSW), jnp.float32),
        pltpu.SemaphoreType.DMA,
        pltpu.SemaphoreType.DMA,
    ],
)
def _sc_gather(td_hbm, ts_hbm, posw_hbm, dst_hbm, src_hbm,
               gd_hbm, gs_hbm, pd_hbm, ps_hbm,
               idxd_v, idxs_v, bufd, bufs, bpd, bps, semd, sems):
    wid = lax.axis_index("s") * SC_CORES + lax.axis_index("c")

    @pl.loop(0, SC_NCHUNK)
    def _(i):
        base = wid * EDGES_PER_WORKER + i * G_CHUNK
        pltpu.sync_copy(dst_hbm.at[pl.ds(base, G_CHUNK)], idxd_v)
        pltpu.sync_copy(src_hbm.at[pl.ds(base, G_CHUNK)], idxs_v)
        cpd = pltpu.async_copy(td_hbm.at[idxd_v], bufd, semd)
        cps = pltpu.async_copy(ts_hbm.at[idxs_v], bufs, sems)
        cpp = pltpu.async_copy(posw_hbm.at[idxd_v], bpd, semd)
        cpq = pltpu.async_copy(posw_hbm.at[idxs_v], bps, sems)
        cpd.wait()
        cps.wait()
        cpp.wait()
        cpq.wait()
        pltpu.sync_copy(bufd, gd_hbm.at[pl.ds(base, G_CHUNK)])
        pltpu.sync_copy(bufs, gs_hbm.at[pl.ds(base, G_CHUNK)])
        pltpu.sync_copy(bpd, pd_hbm.at[pl.ds(base, G_CHUNK)])
        pltpu.sync_copy(bps, ps_hbm.at[pl.ds(base, G_CHUNK)])


@functools.partial(
    pl.kernel,
    mesh=_sc_mesh,
    compiler_params=pltpu.CompilerParams(use_tc_tiling_on_sc=False),
    out_type=[
        jax.ShapeDtypeStruct((SC_CORES, ACC_ROWS, HID), jnp.float32),
        jax.ShapeDtypeStruct((SC_CORES, ACC_ROWS, POSW), jnp.float32),
    ],
    scratch_types=[
        pltpu.VMEM((S_CHUNK,), jnp.int32),
        pltpu.VMEM((S_CHUNK, HID), jnp.float32),
        pltpu.VMEM((S_CHUNK, POSW), jnp.float32),
        pltpu.VMEM_SHARED((ACC_ROWS, HID), jnp.float32),
        pltpu.VMEM_SHARED((ACC_ROWS, POSW), jnp.float32),
    ],
)
def _sc_scatter(mt_hbm, tr_hbm, dst_hbm, zeros_hbm, zeros16_hbm,
                out_hbm, outx_hbm, idx_v, buf, bufx, accum, accx):
    cid = lax.axis_index("c")
    sid = lax.axis_index("s")
    row0 = sid * ROWS_PER_TILE
    pltpu.sync_copy(zeros_hbm, buf)
    pltpu.sync_copy(zeros16_hbm, bufx)
    pltpu.sync_copy(buf.at[pl.ds(0, ROWS_PER_TILE)],
                    accum.at[pl.ds(row0, ROWS_PER_TILE)])
    pltpu.sync_copy(bufx.at[pl.ds(0, ROWS_PER_TILE)],
                    accx.at[pl.ds(row0, ROWS_PER_TILE)])
    plsc.subcore_barrier()

    @pl.loop(0, SCAT_CHUNKS)
    def _(i):
        base = (cid * EDGES_PER_CORE
                + sid * (EDGES_PER_CORE // SC_SUBCORES) + i * S_CHUNK)
        pltpu.sync_copy(dst_hbm.at[pl.ds(base, S_CHUNK)], idx_v)
        pltpu.sync_copy(mt_hbm.at[pl.ds(base, S_CHUNK)], buf)
        pltpu.sync_copy(tr_hbm.at[pl.ds(base, S_CHUNK)], bufx)
        pltpu.sync_copy(buf, accum.at[idx_v], add=True)
        pltpu.sync_copy(bufx, accx.at[idx_v], add=True)

    plsc.subcore_barrier()
    pltpu.sync_copy(accum.at[pl.ds(row0, ROWS_PER_TILE)],
                    buf.at[pl.ds(0, ROWS_PER_TILE)])
    pltpu.sync_copy(buf.at[pl.ds(0, ROWS_PER_TILE)],
                    out_hbm.at[cid, pl.ds(row0, ROWS_PER_TILE)])
    pltpu.sync_copy(accx.at[pl.ds(row0, ROWS_PER_TILE)],
                    bufx.at[pl.ds(0, ROWS_PER_TILE)])
    pltpu.sync_copy(bufx.at[pl.ds(0, ROWS_PER_TILE)],
                    outx_hbm.at[cid, pl.ds(row0, ROWS_PER_TILE)])


def _mid_body(gd_ref, gs_ref, pd_ref, ps_ref, wd_ref, b1_ref, w2_ref, b2_ref,
              wx_ref, bx_ref, m2_ref, trans_ref):
    rel = pd_ref[...] - ps_ref[...]
    d2 = jnp.sum(rel * rel, axis=1, keepdims=True)
    z = gd_ref[...] + gs_ref[...] + d2 * wd_ref[...] + b1_ref[...]
    m1 = z * jax.nn.sigmoid(z)
    y = lax.dot_general(
        m1.astype(jnp.bfloat16), w2_ref[...].astype(jnp.bfloat16),
        (((1,), (0,)), ((), ())), preferred_element_type=jnp.float32)
    y = y + b2_ref[...]
    m2 = y * jax.nn.sigmoid(y)
    m2_ref[...] = m2
    coef = lax.dot_general(
        m2.astype(jnp.bfloat16), wx_ref[...].astype(jnp.bfloat16),
        (((1,), (0,)), ((), ())), preferred_element_type=jnp.float32)
    coef = coef[:, 0:1] + bx_ref[0, 0]
    trans_ref[...] = rel * (coef / (jnp.sqrt(d2) + 1.0))


def _edge_mlp(gd, gs, pd, ps, w_d2, b1, W2, b2, Wx8, bx8):
    full = lambda i: (0, 0)
    return pl.pallas_call(
        _mid_body,
        grid=(E_SLICE // EDGE_BLK,),
        in_specs=[
            pl.BlockSpec((EDGE_BLK, HID), lambda i: (i, 0)),
            pl.BlockSpec((EDGE_BLK, HID), lambda i: (i, 0)),
            pl.BlockSpec((EDGE_BLK, POSW), lambda i: (i, 0)),
            pl.BlockSpec((EDGE_BLK, POSW), lambda i: (i, 0)),
            pl.BlockSpec((1, HID), full),
            pl.BlockSpec((1, HID), full),
            pl.BlockSpec((HID, HID), full),
            pl.BlockSpec((1, HID), full),
            pl.BlockSpec((HID, 8), full),
            pl.BlockSpec((1, 8), full),
        ],
        out_specs=[
            pl.BlockSpec((EDGE_BLK, HID), lambda i: (i, 0)),
            pl.BlockSpec((EDGE_BLK, POSW), lambda i: (i, 0)),
        ],
        out_shape=[
            jax.ShapeDtypeStruct((E_SLICE, HID), jnp.float32),
            jax.ShapeDtypeStruct((E_SLICE, POSW), jnp.float32),
        ],
    )(gd, gs, pd, ps, w_d2, b1, W2, b2, Wx8, bx8)


def _node_body(h_ref, agg_ref, wh1_ref, wh2_ref, bh_ref, h2_ref):
    h = h_ref[...]
    agg = agg_ref[...]
    u = lax.dot_general(
        h.astype(jnp.bfloat16), wh1_ref[...].astype(jnp.bfloat16),
        (((1,), (0,)), ((), ())), preferred_element_type=jnp.float32)
    u = u + lax.dot_general(
        agg.astype(jnp.bfloat16), wh2_ref[...].astype(jnp.bfloat16),
        (((1,), (0,)), ((), ())), preferred_element_type=jnp.float32)
    u = u + bh_ref[...]
    h2_ref[...] = h + u * jax.nn.sigmoid(u)


def _node_update(h, agg, Wh1, Wh2, bh):
    full = lambda i: (0, 0)
    return pl.pallas_call(
        _node_body,
        grid=(N_LIG_ // NODE_BLK,),
        in_specs=[
            pl.BlockSpec((NODE_BLK, HID), lambda i: (i, 0)),
            pl.BlockSpec((NODE_BLK, HID), lambda i: (i, 0)),
            pl.BlockSpec((HID, HID), full),
            pl.BlockSpec((HID, HID), full),
            pl.BlockSpec((1, HID), full),
        ],
        out_specs=pl.BlockSpec((NODE_BLK, HID), lambda i: (i, 0)),
        out_shape=jax.ShapeDtypeStruct((N_LIG_, HID), jnp.float32),
    )(h, agg, Wh1, Wh2, bh)


def kernel(protein_pos, protein_v, init_ligand_pos, W_prot, b_prot, W_lig,
           b_lig, W_e1, b_e1, W_e2, b_e2, W_h, b_h, W_x, b_x, W_v, b_v,
           batch_protein, init_ligand_v, batch_ligand, time_step, edge_index):
    sums = jax.ops.segment_sum(protein_pos, batch_protein,
                               num_segments=NUM_GRAPHS_)
    cnt = jax.ops.segment_sum(jnp.ones((N_PROT_,), jnp.float32),
                              batch_protein, num_segments=NUM_GRAPHS_)
    offset = sums / jnp.maximum(cnt, 1.0)[:, None]
    off_lig = offset[batch_ligand]
    p_pos = protein_pos - offset[batch_protein]
    l_pos = init_ligand_pos - off_lig
    lig_onehot = jax.nn.one_hot(init_ligand_v, 13, dtype=jnp.float32)
    t_feat = (time_step.astype(jnp.float32) / NUM_TIMESTEPS_)[batch_ligand][:, None]
    lig_feat = jnp.concatenate([lig_onehot, t_feat], axis=-1)
    h_prot = protein_v @ W_prot + b_prot
    h_lig = lig_feat @ W_lig + b_lig
    h_prot = jnp.concatenate([h_prot, jnp.zeros((N_PROT_, 1), jnp.float32)], axis=-1)
    h_lig = jnp.concatenate([h_lig, jnp.ones((N_LIG_, 1), jnp.float32)], axis=-1)
    h = jnp.concatenate([h_prot, h_lig], axis=0)
    pos = jnp.concatenate([p_pos, l_pos], axis=0)
    W1d = W_e1[:HID]
    W1s = W_e1[HID:2 * HID]
    w_d2 = W_e1[2 * HID:2 * HID + 1]
    Td = h @ W1d
    Ts = h @ W1s
    src = edge_index[0]
    dst = edge_index[1]
    dst_remap = jnp.where((dst >= N_PROT_) & (dst < N_NODES_),
                          dst - N_PROT_,
                          jnp.int32(LIG_N)).astype(jnp.int32)
    posw = jnp.pad(pos, ((0, 0), (0, POSW - 3)))
    Wx8 = jnp.pad(W_x, ((0, 0), (0, 7)))
    bx8 = jnp.pad(b_x, (0, 7))[None, :]
    zeros_acc = jnp.zeros((S_CHUNK, HID), jnp.float32)
    zeros16 = jnp.zeros((S_CHUNK, POSW), jnp.float32)
    gath = []
    for s in range(K_SLICES):
        sl = slice(s * E_SLICE, (s + 1) * E_SLICE)
        gath.append((_sc_gather(Td, Ts, posw, dst[sl], src[sl]),
                     dst_remap[sl]))
    mids = []
    for (gd, gs, pd, ps), dsl in gath:
        m2, trans = _edge_mlp(gd, gs, pd, ps, w_d2, b_e1[None, :], W_e2,
                              b_e2[None, :], Wx8, bx8)
        mids.append((m2, trans, dsl))
    agg_p = []
    x_p = []
    for m2, trans, dsl in mids:
        parts, partsx = _sc_scatter(m2, trans, dsl, zeros_acc, zeros16)
        agg_p.append(parts)
        x_p.append(partsx)
    parts = agg_p[0] + agg_p[1]
    partsx = x_p[0] + x_p[1]
    agg_l = (parts[0] + parts[1])[:LIG_N]
    x_l = (partsx[0] + partsx[1])[:LIG_N, :3]
    h2l = _node_update(h[N_PROT_:], agg_l, W_h[:HID], W_h[HID:],
                       b_h[None, :])
    pred_ligand_pos = l_pos + x_l + off_lig
    pred_ligand_v = h2l @ W_v + b_v
    return pred_ligand_pos, pred_ligand_v


# speedup vs baseline: 1.7755x; 1.0308x over previous
import functools

import jax
import jax.numpy as jnp
from jax import lax
from jax.experimental import pallas as pl
from jax.experimental.pallas import tpu as pltpu
from jax.experimental.pallas import tpu_sc as plsc

HID = 128
N_PROT_ = 8000
N_LIG_ = 2000
N_NODES_ = N_PROT_ + N_LIG_
N_EDGES_ = 320000
NUM_GRAPHS_ = 16
NUM_TIMESTEPS_ = 1000

EDGE_BLK = 6400
NODE_BLK = 2000

SC_CORES = 2
SC_SUBCORES = 16
SC_WORKERS = SC_CORES * SC_SUBCORES
K_SLICES = 5
E_SLICE = N_EDGES_ // K_SLICES
EDGES_PER_WORKER = E_SLICE // SC_WORKERS
G_CHUNK = 400
SC_NCHUNK = EDGES_PER_WORKER // G_CHUNK
SC_CHUNK = 400
LIG_N = N_LIG_
ACC_ROWS = 2048
ROWS_PER_TILE = ACC_ROWS // SC_SUBCORES
S_CHUNK = 400
EDGES_PER_CORE = E_SLICE // SC_CORES
SCAT_CHUNKS = EDGES_PER_CORE // SC_SUBCORES // S_CHUNK

_sc_mesh = plsc.VectorSubcoreMesh(core_axis_name="c", subcore_axis_name="s")


POSW = 16


@functools.partial(
    pl.kernel,
    mesh=_sc_mesh,
    compiler_params=pltpu.CompilerParams(use_tc_tiling_on_sc=False),
    out_type=[
        jax.ShapeDtypeStruct((E_SLICE, HID), jnp.float32),
        jax.ShapeDtypeStruct((E_SLICE, HID), jnp.float32),
        jax.ShapeDtypeStruct((E_SLICE, POSW), jnp.float32),
        jax.ShapeDtypeStruct((E_SLICE, POSW), jnp.float32),
    ],
    scratch_types=[
        pltpu.VMEM((G_CHUNK,), jnp.int32),
        pltpu.VMEM((G_CHUNK,), jnp.int32),
        pltpu.VMEM((G_CHUNK, HID), jnp.float32),
        pltpu.VMEM((G_CHUNK, HID), jnp.float32),
        pltpu.VMEM((G_CHUNK, POSW), jnp.float32),
        pltpu.VMEM((G_CHUNK, POSW), jnp.float32),
        pltpu.SemaphoreType.DMA,
        pltpu.SemaphoreType.DMA,
    ],
)
def _sc_gather(td_hbm, ts_hbm, posw_hbm, dst_hbm, src_hbm,
               gd_hbm, gs_hbm, pd_hbm, ps_hbm,
               idxd_v, idxs_v, bufd, bufs, bpd, bps, semd, sems):
    wid = lax.axis_index("s") * SC_CORES + lax.axis_index("c")

    @pl.loop(0, SC_NCHUNK)
    def _(i):
        base = wid * EDGES_PER_WORKER + i * G_CHUNK
        pltpu.sync_copy(dst_hbm.at[pl.ds(base, G_CHUNK)], idxd_v)
        pltpu.sync_copy(src_hbm.at[pl.ds(base, G_CHUNK)], idxs_v)
        cpd = pltpu.async_copy(td_hbm.at[idxd_v], bufd, semd)
        cps = pltpu.async_copy(ts_hbm.at[idxs_v], bufs, sems)
        cpp = pltpu.async_copy(posw_hbm.at[idxd_v], bpd, semd)
        cpq = pltpu.async_copy(posw_hbm.at[idxs_v], bps, sems)
        cpd.wait()
        cps.wait()
        cpp.wait()
        cpq.wait()
        pltpu.sync_copy(bufd, gd_hbm.at[pl.ds(base, G_CHUNK)])
        pltpu.sync_copy(bufs, gs_hbm.at[pl.ds(base, G_CHUNK)])
        pltpu.sync_copy(bpd, pd_hbm.at[pl.ds(base, G_CHUNK)])
        pltpu.sync_copy(bps, ps_hbm.at[pl.ds(base, G_CHUNK)])


@functools.partial(
    pl.kernel,
    mesh=_sc_mesh,
    compiler_params=pltpu.CompilerParams(use_tc_tiling_on_sc=False),
    out_type=[
        jax.ShapeDtypeStruct((SC_CORES, ACC_ROWS, HID), jnp.float32),
        jax.ShapeDtypeStruct((SC_CORES, ACC_ROWS, POSW), jnp.float32),
    ],
    scratch_types=[
        pltpu.VMEM((S_CHUNK,), jnp.int32),
        pltpu.VMEM((S_CHUNK, HID), jnp.float32),
        pltpu.VMEM((S_CHUNK, POSW), jnp.float32),
        pltpu.VMEM_SHARED((ACC_ROWS, HID), jnp.float32),
        pltpu.VMEM_SHARED((ACC_ROWS, POSW), jnp.float32),
    ],
)
def _sc_scatter(mt_hbm, tr_hbm, dst_hbm, zeros_hbm, zeros16_hbm,
                out_hbm, outx_hbm, idx_v, buf, bufx, accum, accx):
    cid = lax.axis_index("c")
    sid = lax.axis_index("s")
    row0 = sid * ROWS_PER_TILE
    pltpu.sync_copy(zeros_hbm, buf)
    pltpu.sync_copy(zeros16_hbm, bufx)
    pltpu.sync_copy(buf.at[pl.ds(0, ROWS_PER_TILE)],
                    accum.at[pl.ds(row0, ROWS_PER_TILE)])
    pltpu.sync_copy(bufx.at[pl.ds(0, ROWS_PER_TILE)],
                    accx.at[pl.ds(row0, ROWS_PER_TILE)])
    plsc.subcore_barrier()

    @pl.loop(0, SCAT_CHUNKS)
    def _(i):
        base = (cid * EDGES_PER_CORE
                + sid * (EDGES_PER_CORE // SC_SUBCORES) + i * S_CHUNK)
        pltpu.sync_copy(dst_hbm.at[pl.ds(base, S_CHUNK)], idx_v)
        pltpu.sync_copy(mt_hbm.at[pl.ds(base, S_CHUNK)], buf)
        pltpu.sync_copy(tr_hbm.at[pl.ds(base, S_CHUNK)], bufx)
        pltpu.sync_copy(buf, accum.at[idx_v], add=True)
        pltpu.sync_copy(bufx, accx.at[idx_v], add=True)

    plsc.subcore_barrier()
    pltpu.sync_copy(accum.at[pl.ds(row0, ROWS_PER_TILE)],
                    buf.at[pl.ds(0, ROWS_PER_TILE)])
    pltpu.sync_copy(buf.at[pl.ds(0, ROWS_PER_TILE)],
                    out_hbm.at[cid, pl.ds(row0, ROWS_PER_TILE)])
    pltpu.sync_copy(accx.at[pl.ds(row0, ROWS_PER_TILE)],
                    bufx.at[pl.ds(0, ROWS_PER_TILE)])
    pltpu.sync_copy(bufx.at[pl.ds(0, ROWS_PER_TILE)],
                    outx_hbm.at[cid, pl.ds(row0, ROWS_PER_TILE)])


def _mid_body(gd_ref, gs_ref, pd_ref, ps_ref, wd_ref, b1_ref, w2_ref, b2_ref,
              wx_ref, bx_ref, m2_ref, trans_ref):
    rel = pd_ref[...] - ps_ref[...]
    d2 = jnp.sum(rel * rel, axis=1, keepdims=True)
    z = gd_ref[...] + gs_ref[...] + d2 * wd_ref[...] + b1_ref[...]
    m1 = z * jax.nn.sigmoid(z)
    y = lax.dot_general(
        m1.astype(jnp.bfloat16), w2_ref[...].astype(jnp.bfloat16),
        (((1,), (0,)), ((), ())), preferred_element_type=jnp.float32)
    y = y + b2_ref[...]
    m2 = y * jax.nn.sigmoid(y)
    m2_ref[...] = m2
    coef = lax.dot_general(
        m2.astype(jnp.bfloat16), wx_ref[...].astype(jnp.bfloat16),
        (((1,), (0,)), ((), ())), preferred_element_type=jnp.float32)
    coef = coef[:, 0:1] + bx_ref[0, 0]
    trans_ref[...] = rel * (coef / (jnp.sqrt(d2) + 1.0))


def _edge_mlp(gd, gs, pd, ps, w_d2, b1, W2, b2, Wx8, bx8):
    full = lambda i: (0, 0)
    return pl.pallas_call(
        _mid_body,
        grid=(E_SLICE // EDGE_BLK,),
        in_specs=[
            pl.BlockSpec((EDGE_BLK, HID), lambda i: (i, 0)),
            pl.BlockSpec((EDGE_BLK, HID), lambda i: (i, 0)),
            pl.BlockSpec((EDGE_BLK, POSW), lambda i: (i, 0)),
            pl.BlockSpec((EDGE_BLK, POSW), lambda i: (i, 0)),
            pl.BlockSpec((1, HID), full),
            pl.BlockSpec((1, HID), full),
            pl.BlockSpec((HID, HID), full),
            pl.BlockSpec((1, HID), full),
            pl.BlockSpec((HID, 8), full),
            pl.BlockSpec((1, 8), full),
        ],
        out_specs=[
            pl.BlockSpec((EDGE_BLK, HID), lambda i: (i, 0)),
            pl.BlockSpec((EDGE_BLK, POSW), lambda i: (i, 0)),
        ],
        out_shape=[
            jax.ShapeDtypeStruct((E_SLICE, HID), jnp.float32),
            jax.ShapeDtypeStruct((E_SLICE, POSW), jnp.float32),
        ],
    )(gd, gs, pd, ps, w_d2, b1, W2, b2, Wx8, bx8)


def _node_body(h_ref, agg_ref, wh1_ref, wh2_ref, bh_ref, h2_ref):
    h = h_ref[...]
    agg = agg_ref[...]
    u = lax.dot_general(
        h.astype(jnp.bfloat16), wh1_ref[...].astype(jnp.bfloat16),
        (((1,), (0,)), ((), ())), preferred_element_type=jnp.float32)
    u = u + lax.dot_general(
        agg.astype(jnp.bfloat16), wh2_ref[...].astype(jnp.bfloat16),
        (((1,), (0,)), ((), ())), preferred_element_type=jnp.float32)
    u = u + bh_ref[...]
    h2_ref[...] = h + u * jax.nn.sigmoid(u)


def _node_update(h, agg, Wh1, Wh2, bh):
    full = lambda i: (0, 0)
    return pl.pallas_call(
        _node_body,
        grid=(N_LIG_ // NODE_BLK,),
        in_specs=[
            pl.BlockSpec((NODE_BLK, HID), lambda i: (i, 0)),
            pl.BlockSpec((NODE_BLK, HID), lambda i: (i, 0)),
            pl.BlockSpec((HID, HID), full),
            pl.BlockSpec((HID, HID), full),
            pl.BlockSpec((1, HID), full),
        ],
        out_specs=pl.BlockSpec((NODE_BLK, HID), lambda i: (i, 0)),
        out_shape=jax.ShapeDtypeStruct((N_LIG_, HID), jnp.float32),
    )(h, agg, Wh1, Wh2, bh)


def kernel(protein_pos, protein_v, init_ligand_pos, W_prot, b_prot, W_lig,
           b_lig, W_e1, b_e1, W_e2, b_e2, W_h, b_h, W_x, b_x, W_v, b_v,
           batch_protein, init_ligand_v, batch_ligand, time_step, edge_index):
    sums = jax.ops.segment_sum(protein_pos, batch_protein,
                               num_segments=NUM_GRAPHS_)
    cnt = jax.ops.segment_sum(jnp.ones((N_PROT_,), jnp.float32),
                              batch_protein, num_segments=NUM_GRAPHS_)
    offset = sums / jnp.maximum(cnt, 1.0)[:, None]
    off_lig = offset[batch_ligand]
    p_pos = protein_pos - offset[batch_protein]
    l_pos = init_ligand_pos - off_lig
    lig_onehot = jax.nn.one_hot(init_ligand_v, 13, dtype=jnp.float32)
    t_feat = (time_step.astype(jnp.float32) / NUM_TIMESTEPS_)[batch_ligand][:, None]
    lig_feat = jnp.concatenate([lig_onehot, t_feat], axis=-1)
    h_prot = protein_v @ W_prot + b_prot
    h_lig = lig_feat @ W_lig + b_lig
    h_prot = jnp.concatenate([h_prot, jnp.zeros((N_PROT_, 1), jnp.float32)], axis=-1)
    h_lig = jnp.concatenate([h_lig, jnp.ones((N_LIG_, 1), jnp.float32)], axis=-1)
    h = jnp.concatenate([h_prot, h_lig], axis=0)
    pos = jnp.concatenate([p_pos, l_pos], axis=0)
    W1d = W_e1[:HID]
    W1s = W_e1[HID:2 * HID]
    w_d2 = W_e1[2 * HID:2 * HID + 1]
    Td = h @ W1d
    Ts = h @ W1s
    src = edge_index[0]
    dst = edge_index[1]
    dst_remap = jnp.where((dst >= N_PROT_) & (dst < N_NODES_),
                          dst - N_PROT_,
                          jnp.int32(LIG_N)).astype(jnp.int32)
    posw = jnp.pad(pos, ((0, 0), (0, POSW - 3)))
    Wx8 = jnp.pad(W_x, ((0, 0), (0, 7)))
    bx8 = jnp.pad(b_x, (0, 7))[None, :]
    zeros_acc = jnp.zeros((S_CHUNK, HID), jnp.float32)
    zeros16 = jnp.zeros((S_CHUNK, POSW), jnp.float32)
    gath = []
    for s in range(K_SLICES):
        sl = slice(s * E_SLICE, (s + 1) * E_SLICE)
        gath.append((_sc_gather(Td, Ts, posw, dst[sl], src[sl]),
                     dst_remap[sl]))
    mids = []
    for (gd, gs, pd, ps), dsl in gath:
        m2, trans = _edge_mlp(gd, gs, pd, ps, w_d2, b_e1[None, :], W_e2,
                              b_e2[None, :], Wx8, bx8)
        mids.append((m2, trans, dsl))
    agg_p = []
    x_p = []
    for m2, trans, dsl in mids:
        parts, partsx = _sc_scatter(m2, trans, dsl, zeros_acc, zeros16)
        agg_p.append(parts)
        x_p.append(partsx)
    parts = sum(agg_p)
    partsx = sum(x_p)
    agg_l = (parts[0] + parts[1])[:LIG_N]
    x_l = (partsx[0] + partsx[1])[:LIG_N, :3]
    h2l = _node_update(h[N_PROT_:], agg_l, W_h[:HID], W_h[HID:],
                       b_h[None, :])
    pred_ligand_pos = l_pos + x_l + off_lig
    pred_ligand_v = h2l @ W_v + b_v
    return pred_ligand_pos, pred_ligand_v


# speedup vs baseline: 1.8452x; 1.0392x over previous
import functools

import jax
import jax.numpy as jnp
from jax import lax
from jax.experimental import pallas as pl
from jax.experimental.pallas import tpu as pltpu
from jax.experimental.pallas import tpu_sc as plsc

HID = 128
N_PROT_ = 8000
N_LIG_ = 2000
N_NODES_ = N_PROT_ + N_LIG_
N_EDGES_ = 320000
NUM_GRAPHS_ = 16
NUM_TIMESTEPS_ = 1000

EDGE_BLK = 6400
NODE_BLK = 2000

SC_CORES = 2
SC_SUBCORES = 16
SC_WORKERS = SC_CORES * SC_SUBCORES
K_SLICES = 5
E_SLICE = N_EDGES_ // K_SLICES
EDGES_PER_WORKER = E_SLICE // SC_WORKERS
G_CHUNK = 400
SC_NCHUNK = EDGES_PER_WORKER // G_CHUNK
SC_CHUNK = 400
LIG_N = N_LIG_
ACC_ROWS = 2048
ROWS_PER_TILE = ACC_ROWS // SC_SUBCORES
S_CHUNK = 400
EDGES_PER_CORE = E_SLICE // SC_CORES
SCAT_CHUNKS = EDGES_PER_CORE // SC_SUBCORES // S_CHUNK

_sc_mesh = plsc.VectorSubcoreMesh(core_axis_name="c", subcore_axis_name="s")


POSW = 16


@functools.partial(
    pl.kernel,
    mesh=_sc_mesh,
    compiler_params=pltpu.CompilerParams(use_tc_tiling_on_sc=False),
    out_type=[
        jax.ShapeDtypeStruct((E_SLICE, HID), jnp.float32),
        jax.ShapeDtypeStruct((E_SLICE, HID), jnp.float32),
        jax.ShapeDtypeStruct((E_SLICE, POSW), jnp.float32),
        jax.ShapeDtypeStruct((E_SLICE, POSW), jnp.float32),
    ],
    scratch_types=[
        pltpu.VMEM((G_CHUNK,), jnp.int32),
        pltpu.VMEM((G_CHUNK,), jnp.int32),
        pltpu.VMEM((G_CHUNK, HID), jnp.float32),
        pltpu.VMEM((G_CHUNK, HID), jnp.float32),
        pltpu.VMEM((G_CHUNK, POSW), jnp.float32),
        pltpu.VMEM((G_CHUNK, POSW), jnp.float32),
        pltpu.SemaphoreType.DMA,
        pltpu.SemaphoreType.DMA,
    ],
)
def _sc_gather(td_hbm, ts_hbm, posw_hbm, dst_hbm, src_hbm,
               gd_hbm, gs_hbm, pd_hbm, ps_hbm,
               idxd_v, idxs_v, bufd, bufs, bpd, bps, semd, sems):
    wid = lax.axis_index("s") * SC_CORES + lax.axis_index("c")

    @pl.loop(0, SC_NCHUNK)
    def _(i):
        base = wid * EDGES_PER_WORKER + i * G_CHUNK
        pltpu.sync_copy(dst_hbm.at[pl.ds(base, G_CHUNK)], idxd_v)
        pltpu.sync_copy(src_hbm.at[pl.ds(base, G_CHUNK)], idxs_v)
        cpd = pltpu.async_copy(td_hbm.at[idxd_v], bufd, semd)
        cps = pltpu.async_copy(ts_hbm.at[idxs_v], bufs, sems)
        cpp = pltpu.async_copy(posw_hbm.at[idxd_v], bpd, semd)
        cpq = pltpu.async_copy(posw_hbm.at[idxs_v], bps, sems)
        cpd.wait()
        cps.wait()
        cpp.wait()
        cpq.wait()
        pltpu.sync_copy(bufd, gd_hbm.at[pl.ds(base, G_CHUNK)])
        pltpu.sync_copy(bufs, gs_hbm.at[pl.ds(base, G_CHUNK)])
        pltpu.sync_copy(bpd, pd_hbm.at[pl.ds(base, G_CHUNK)])
        pltpu.sync_copy(bps, ps_hbm.at[pl.ds(base, G_CHUNK)])


@functools.partial(
    pl.kernel,
    mesh=_sc_mesh,
    compiler_params=pltpu.CompilerParams(use_tc_tiling_on_sc=False),
    out_type=[
        jax.ShapeDtypeStruct((SC_CORES, ACC_ROWS, HID), jnp.float32),
        jax.ShapeDtypeStruct((SC_CORES, ACC_ROWS, POSW), jnp.float32),
    ],
    scratch_types=[
        pltpu.VMEM((S_CHUNK,), jnp.int32),
        pltpu.VMEM((S_CHUNK, HID), jnp.float32),
        pltpu.VMEM((S_CHUNK, POSW), jnp.float32),
        pltpu.VMEM_SHARED((ACC_ROWS, HID), jnp.float32),
        pltpu.VMEM_SHARED((ACC_ROWS, POSW), jnp.float32),
    ],
)
def _sc_scatter(mt_hbm, tr_hbm, dst_hbm, zeros_hbm, zeros16_hbm,
                out_hbm, outx_hbm, idx_v, buf, bufx, accum, accx):
    cid = lax.axis_index("c")
    sid = lax.axis_index("s")
    row0 = sid * ROWS_PER_TILE
    pltpu.sync_copy(zeros_hbm, buf)
    pltpu.sync_copy(zeros16_hbm, bufx)
    pltpu.sync_copy(buf.at[pl.ds(0, ROWS_PER_TILE)],
                    accum.at[pl.ds(row0, ROWS_PER_TILE)])
    pltpu.sync_copy(bufx.at[pl.ds(0, ROWS_PER_TILE)],
                    accx.at[pl.ds(row0, ROWS_PER_TILE)])
    plsc.subcore_barrier()

    @pl.loop(0, SCAT_CHUNKS)
    def _(i):
        base = (cid * EDGES_PER_CORE
                + sid * (EDGES_PER_CORE // SC_SUBCORES) + i * S_CHUNK)
        pltpu.sync_copy(dst_hbm.at[pl.ds(base, S_CHUNK)], idx_v)
        pltpu.sync_copy(mt_hbm.at[pl.ds(base, S_CHUNK)], buf)
        pltpu.sync_copy(tr_hbm.at[pl.ds(base, S_CHUNK)], bufx)
        pltpu.sync_copy(buf, accum.at[idx_v], add=True)
        pltpu.sync_copy(bufx, accx.at[idx_v], add=True)

    plsc.subcore_barrier()
    pltpu.sync_copy(accum.at[pl.ds(row0, ROWS_PER_TILE)],
                    buf.at[pl.ds(0, ROWS_PER_TILE)])
    pltpu.sync_copy(buf.at[pl.ds(0, ROWS_PER_TILE)],
                    out_hbm.at[cid, pl.ds(row0, ROWS_PER_TILE)])
    pltpu.sync_copy(accx.at[pl.ds(row0, ROWS_PER_TILE)],
                    bufx.at[pl.ds(0, ROWS_PER_TILE)])
    pltpu.sync_copy(bufx.at[pl.ds(0, ROWS_PER_TILE)],
                    outx_hbm.at[cid, pl.ds(row0, ROWS_PER_TILE)])


def _mid_body(gd_ref, gs_ref, pd_ref, ps_ref, wd_ref, b1_ref, w2_ref, b2_ref,
              wx_ref, bx_ref, m2_ref, trans_ref):
    rel = pd_ref[...] - ps_ref[...]
    d2 = jnp.sum(rel * rel, axis=1, keepdims=True)
    z = gd_ref[...] + gs_ref[...] + d2 * wd_ref[...] + b1_ref[...]
    m1 = z * jax.nn.sigmoid(z)
    y = lax.dot_general(
        m1.astype(jnp.bfloat16), w2_ref[...].astype(jnp.bfloat16),
        (((1,), (0,)), ((), ())), preferred_element_type=jnp.float32)
    y = y + b2_ref[...]
    m2 = y * jax.nn.sigmoid(y)
    m2_ref[...] = m2
    coef = lax.dot_general(
        m2.astype(jnp.bfloat16), wx_ref[...].astype(jnp.bfloat16),
        (((1,), (0,)), ((), ())), preferred_element_type=jnp.float32)
    coef = coef[:, 0:1] + bx_ref[0, 0]
    trans_ref[...] = rel * (coef / (jnp.sqrt(d2) + 1.0))


def _edge_mlp(gd, gs, pd, ps, w_d2, b1, W2, b2, Wx8, bx8):
    full = lambda i: (0, 0)
    return pl.pallas_call(
        _mid_body,
        grid=(E_SLICE // EDGE_BLK,),
        in_specs=[
            pl.BlockSpec((EDGE_BLK, HID), lambda i: (i, 0)),
            pl.BlockSpec((EDGE_BLK, HID), lambda i: (i, 0)),
            pl.BlockSpec((EDGE_BLK, POSW), lambda i: (i, 0)),
            pl.BlockSpec((EDGE_BLK, POSW), lambda i: (i, 0)),
            pl.BlockSpec((1, HID), full),
            pl.BlockSpec((1, HID), full),
            pl.BlockSpec((HID, HID), full),
            pl.BlockSpec((1, HID), full),
            pl.BlockSpec((HID, 8), full),
            pl.BlockSpec((1, 8), full),
        ],
        out_specs=[
            pl.BlockSpec((EDGE_BLK, HID), lambda i: (i, 0)),
            pl.BlockSpec((EDGE_BLK, POSW), lambda i: (i, 0)),
        ],
        out_shape=[
            jax.ShapeDtypeStruct((E_SLICE, HID), jnp.float32),
            jax.ShapeDtypeStruct((E_SLICE, POSW), jnp.float32),
        ],
    )(gd, gs, pd, ps, w_d2, b1, W2, b2, Wx8, bx8)


def _node_body(h_ref, agg_ref, wh1_ref, wh2_ref, bh_ref, h2_ref):
    h = h_ref[...]
    agg = agg_ref[...]
    u = lax.dot_general(
        h.astype(jnp.bfloat16), wh1_ref[...].astype(jnp.bfloat16),
        (((1,), (0,)), ((), ())), preferred_element_type=jnp.float32)
    u = u + lax.dot_general(
        agg.astype(jnp.bfloat16), wh2_ref[...].astype(jnp.bfloat16),
        (((1,), (0,)), ((), ())), preferred_element_type=jnp.float32)
    u = u + bh_ref[...]
    h2_ref[...] = h + u * jax.nn.sigmoid(u)


def _node_update(h, agg, Wh1, Wh2, bh):
    full = lambda i: (0, 0)
    return pl.pallas_call(
        _node_body,
        grid=(N_LIG_ // NODE_BLK,),
        in_specs=[
            pl.BlockSpec((NODE_BLK, HID), lambda i: (i, 0)),
            pl.BlockSpec((NODE_BLK, HID), lambda i: (i, 0)),
            pl.BlockSpec((HID, HID), full),
            pl.BlockSpec((HID, HID), full),
            pl.BlockSpec((1, HID), full),
        ],
        out_specs=pl.BlockSpec((NODE_BLK, HID), lambda i: (i, 0)),
        out_shape=jax.ShapeDtypeStruct((N_LIG_, HID), jnp.float32),
    )(h, agg, Wh1, Wh2, bh)


def kernel(protein_pos, protein_v, init_ligand_pos, W_prot, b_prot, W_lig,
           b_lig, W_e1, b_e1, W_e2, b_e2, W_h, b_h, W_x, b_x, W_v, b_v,
           batch_protein, init_ligand_v, batch_ligand, time_step, edge_index):
    ones = jnp.ones((N_PROT_, 1), jnp.float32)
    sums4 = jax.ops.segment_sum(jnp.concatenate([protein_pos, ones], axis=1),
                                batch_protein, num_segments=NUM_GRAPHS_)
    offset = sums4[:, :3] / jnp.maximum(sums4[:, 3:4], 1.0)
    oh_p = jax.nn.one_hot(batch_protein, NUM_GRAPHS_, dtype=jnp.float32)
    oh_l = jax.nn.one_hot(batch_ligand, NUM_GRAPHS_, dtype=jnp.float32)
    off_lig = oh_l @ offset
    p_pos = protein_pos - oh_p @ offset
    l_pos = init_ligand_pos - off_lig
    lig_onehot = jax.nn.one_hot(init_ligand_v, 13, dtype=jnp.float32)
    t_feat = oh_l @ (time_step.astype(jnp.float32) / NUM_TIMESTEPS_)[:, None]
    lig_feat = jnp.concatenate([lig_onehot, t_feat], axis=-1)
    h_lig = jnp.concatenate([lig_feat @ W_lig + b_lig,
                             jnp.ones((N_LIG_, 1), jnp.float32)], axis=-1)
    pos = jnp.concatenate([p_pos, l_pos], axis=0)
    W1d = W_e1[:HID]
    W1s = W_e1[HID:2 * HID]
    w_d2 = W_e1[2 * HID:2 * HID + 1]
    emb = HID - 1
    Td = jnp.concatenate(
        [protein_v @ (W_prot @ W1d[:emb]) + b_prot @ W1d[:emb],
         lig_feat @ (W_lig @ W1d[:emb]) + b_lig @ W1d[:emb] + W1d[emb]],
        axis=0)
    Ts = jnp.concatenate(
        [protein_v @ (W_prot @ W1s[:emb]) + b_prot @ W1s[:emb],
         lig_feat @ (W_lig @ W1s[:emb]) + b_lig @ W1s[:emb] + W1s[emb]],
        axis=0)
    src = edge_index[0]
    dst = edge_index[1]
    dst_remap = jnp.where((dst >= N_PROT_) & (dst < N_NODES_),
                          dst - N_PROT_,
                          jnp.int32(LIG_N)).astype(jnp.int32)
    posw = jnp.pad(pos, ((0, 0), (0, POSW - 3)))
    Wx8 = jnp.pad(W_x, ((0, 0), (0, 7)))
    bx8 = jnp.pad(b_x, (0, 7))[None, :]
    zeros_acc = jnp.zeros((S_CHUNK, HID), jnp.float32)
    zeros16 = jnp.zeros((S_CHUNK, POSW), jnp.float32)
    gath = []
    for s in range(K_SLICES):
        sl = slice(s * E_SLICE, (s + 1) * E_SLICE)
        gath.append((_sc_gather(Td, Ts, posw, dst[sl], src[sl]),
                     dst_remap[sl]))
    mids = []
    for (gd, gs, pd, ps), dsl in gath:
        m2, trans = _edge_mlp(gd, gs, pd, ps, w_d2, b_e1[None, :], W_e2,
                              b_e2[None, :], Wx8, bx8)
        mids.append((m2, trans, dsl))
    agg_p = []
    x_p = []
    for m2, trans, dsl in mids:
        parts, partsx = _sc_scatter(m2, trans, dsl, zeros_acc, zeros16)
        agg_p.append(parts)
        x_p.append(partsx)
    parts = sum(agg_p)
    partsx = sum(x_p)
    agg_l = (parts[0] + parts[1])[:LIG_N]
    x_l = (partsx[0] + partsx[1])[:LIG_N, :3]
    h2l = _node_update(h_lig, agg_l, W_h[:HID], W_h[HID:],
                       b_h[None, :])
    pred_ligand_pos = l_pos + x_l + off_lig
    pred_ligand_v = h2l @ W_v + b_v
    return pred_ligand_pos, pred_ligand_v


# speedup vs baseline: 1.8469x; 1.0010x over previous
import functools

import jax
import jax.numpy as jnp
from jax import lax
from jax.experimental import pallas as pl
from jax.experimental.pallas import tpu as pltpu
from jax.experimental.pallas import tpu_sc as plsc

HID = 128
N_PROT_ = 8000
N_LIG_ = 2000
N_NODES_ = N_PROT_ + N_LIG_
N_EDGES_ = 320000
NUM_GRAPHS_ = 16
NUM_TIMESTEPS_ = 1000

EDGE_BLK = 6400
NODE_BLK = 2000

SC_CORES = 2
SC_SUBCORES = 16
SC_WORKERS = SC_CORES * SC_SUBCORES
K_SLICES = 5
E_SLICE = N_EDGES_ // K_SLICES
EDGES_PER_WORKER = E_SLICE // SC_WORKERS
G_CHUNK = 400
SC_NCHUNK = EDGES_PER_WORKER // G_CHUNK
LIG_N = N_LIG_
ACC_ROWS = 2048
ROWS_PER_TILE = ACC_ROWS // SC_SUBCORES
S_CHUNK = 400
EDGES_PER_CORE = E_SLICE // SC_CORES
SCAT_CHUNKS = EDGES_PER_CORE // SC_SUBCORES // S_CHUNK

_sc_mesh = plsc.VectorSubcoreMesh(core_axis_name="c", subcore_axis_name="s")


POSW = 16


@functools.partial(
    pl.kernel,
    mesh=_sc_mesh,
    compiler_params=pltpu.CompilerParams(use_tc_tiling_on_sc=False),
    out_type=[
        jax.ShapeDtypeStruct((E_SLICE, HID), jnp.float32),
        jax.ShapeDtypeStruct((E_SLICE, HID), jnp.float32),
        jax.ShapeDtypeStruct((E_SLICE, POSW), jnp.float32),
        jax.ShapeDtypeStruct((E_SLICE, POSW), jnp.float32),
    ],
    scratch_types=[
        pltpu.VMEM((G_CHUNK,), jnp.int32),
        pltpu.VMEM((G_CHUNK,), jnp.int32),
        pltpu.VMEM((G_CHUNK, HID), jnp.float32),
        pltpu.VMEM((G_CHUNK, HID), jnp.float32),
        pltpu.VMEM((G_CHUNK, POSW), jnp.float32),
        pltpu.VMEM((G_CHUNK, POSW), jnp.float32),
        pltpu.SemaphoreType.DMA,
        pltpu.SemaphoreType.DMA,
    ],
)
def _sc_gather(td_hbm, ts_hbm, posw_hbm, dst_hbm, src_hbm,
               gd_hbm, gs_hbm, pd_hbm, ps_hbm,
               idxd_v, idxs_v, bufd, bufs, bpd, bps, semd, sems):
    wid = lax.axis_index("s") * SC_CORES + lax.axis_index("c")

    @pl.loop(0, SC_NCHUNK)
    def _(i):
        base = wid * EDGES_PER_WORKER + i * G_CHUNK
        pltpu.sync_copy(dst_hbm.at[pl.ds(base, G_CHUNK)], idxd_v)
        pltpu.sync_copy(src_hbm.at[pl.ds(base, G_CHUNK)], idxs_v)
        cpd = pltpu.async_copy(td_hbm.at[idxd_v], bufd, semd)
        cps = pltpu.async_copy(ts_hbm.at[idxs_v], bufs, sems)
        cpp = pltpu.async_copy(posw_hbm.at[idxd_v], bpd, semd)
        cpq = pltpu.async_copy(posw_hbm.at[idxs_v], bps, sems)
        cpd.wait()
        cps.wait()
        cpp.wait()
        cpq.wait()
        pltpu.sync_copy(bufd, gd_hbm.at[pl.ds(base, G_CHUNK)])
        pltpu.sync_copy(bufs, gs_hbm.at[pl.ds(base, G_CHUNK)])
        pltpu.sync_copy(bpd, pd_hbm.at[pl.ds(base, G_CHUNK)])
        pltpu.sync_copy(bps, ps_hbm.at[pl.ds(base, G_CHUNK)])


@functools.partial(
    pl.kernel,
    mesh=_sc_mesh,
    compiler_params=pltpu.CompilerParams(use_tc_tiling_on_sc=False),
    out_type=[
        jax.ShapeDtypeStruct((SC_CORES, ACC_ROWS, HID), jnp.float32),
        jax.ShapeDtypeStruct((SC_CORES, ACC_ROWS, POSW), jnp.float32),
    ],
    scratch_types=[
        pltpu.VMEM((S_CHUNK,), jnp.int32),
        pltpu.VMEM((S_CHUNK, HID), jnp.float32),
        pltpu.VMEM((S_CHUNK, POSW), jnp.float32),
        pltpu.VMEM_SHARED((ACC_ROWS, HID), jnp.float32),
        pltpu.VMEM_SHARED((ACC_ROWS, POSW), jnp.float32),
    ],
)
def _sc_scatter(mt_hbm, tr_hbm, dst_hbm, zeros_hbm, zeros16_hbm,
                out_hbm, outx_hbm, idx_v, buf, bufx, accum, accx):
    cid = lax.axis_index("c")
    sid = lax.axis_index("s")
    row0 = sid * ROWS_PER_TILE
    pltpu.sync_copy(zeros_hbm, buf)
    pltpu.sync_copy(zeros16_hbm, bufx)
    pltpu.sync_copy(buf.at[pl.ds(0, ROWS_PER_TILE)],
                    accum.at[pl.ds(row0, ROWS_PER_TILE)])
    pltpu.sync_copy(bufx.at[pl.ds(0, ROWS_PER_TILE)],
                    accx.at[pl.ds(row0, ROWS_PER_TILE)])
    plsc.subcore_barrier()

    @pl.loop(0, SCAT_CHUNKS)
    def _(i):
        base = (cid * EDGES_PER_CORE
                + sid * (EDGES_PER_CORE // SC_SUBCORES) + i * S_CHUNK)
        pltpu.sync_copy(dst_hbm.at[pl.ds(base, S_CHUNK)], idx_v)
        pltpu.sync_copy(mt_hbm.at[pl.ds(base, S_CHUNK)], buf)
        pltpu.sync_copy(tr_hbm.at[pl.ds(base, S_CHUNK)], bufx)
        pltpu.sync_copy(buf, accum.at[idx_v], add=True)
        pltpu.sync_copy(bufx, accx.at[idx_v], add=True)

    plsc.subcore_barrier()
    pltpu.sync_copy(accum.at[pl.ds(row0, ROWS_PER_TILE)],
                    buf.at[pl.ds(0, ROWS_PER_TILE)])
    pltpu.sync_copy(buf.at[pl.ds(0, ROWS_PER_TILE)],
                    out_hbm.at[cid, pl.ds(row0, ROWS_PER_TILE)])
    pltpu.sync_copy(accx.at[pl.ds(row0, ROWS_PER_TILE)],
                    bufx.at[pl.ds(0, ROWS_PER_TILE)])
    pltpu.sync_copy(bufx.at[pl.ds(0, ROWS_PER_TILE)],
                    outx_hbm.at[cid, pl.ds(row0, ROWS_PER_TILE)])


def _mid_body(gd_ref, gs_ref, pd_ref, ps_ref, wd_ref, b1_ref, w2_ref, b2_ref,
              wx_ref, bx_ref, m2_ref, trans_ref):
    rel = pd_ref[...] - ps_ref[...]
    d2 = jnp.sum(rel * rel, axis=1, keepdims=True)
    z = gd_ref[...] + gs_ref[...] + d2 * wd_ref[...] + b1_ref[...]
    m1 = z * jax.nn.sigmoid(z)
    y = lax.dot_general(
        m1.astype(jnp.bfloat16), w2_ref[...].astype(jnp.bfloat16),
        (((1,), (0,)), ((), ())), preferred_element_type=jnp.float32)
    y = y + b2_ref[...]
    m2 = y * jax.nn.sigmoid(y)
    m2_ref[...] = m2
    coef = lax.dot_general(
        m2.astype(jnp.bfloat16), wx_ref[...].astype(jnp.bfloat16),
        (((1,), (0,)), ((), ())), preferred_element_type=jnp.float32)
    coef = coef[:, 0:1] + bx_ref[0, 0]
    trans_ref[...] = rel * (coef / (jnp.sqrt(d2) + 1.0))


def _edge_mlp(gd, gs, pd, ps, w_d2, b1, W2, b2, Wx8, bx8):
    full = lambda i: (0, 0)
    return pl.pallas_call(
        _mid_body,
        grid=(E_SLICE // EDGE_BLK,),
        in_specs=[
            pl.BlockSpec((EDGE_BLK, HID), lambda i: (i, 0)),
            pl.BlockSpec((EDGE_BLK, HID), lambda i: (i, 0)),
            pl.BlockSpec((EDGE_BLK, POSW), lambda i: (i, 0)),
            pl.BlockSpec((EDGE_BLK, POSW), lambda i: (i, 0)),
            pl.BlockSpec((1, HID), full),
            pl.BlockSpec((1, HID), full),
            pl.BlockSpec((HID, HID), full),
            pl.BlockSpec((1, HID), full),
            pl.BlockSpec((HID, 8), full),
            pl.BlockSpec((1, 8), full),
        ],
        out_specs=[
            pl.BlockSpec((EDGE_BLK, HID), lambda i: (i, 0)),
            pl.BlockSpec((EDGE_BLK, POSW), lambda i: (i, 0)),
        ],
        out_shape=[
            jax.ShapeDtypeStruct((E_SLICE, HID), jnp.float32),
            jax.ShapeDtypeStruct((E_SLICE, POSW), jnp.float32),
        ],
    )(gd, gs, pd, ps, w_d2, b1, W2, b2, Wx8, bx8)


def _node_body(h_ref, agg_ref, wh1_ref, wh2_ref, bh_ref, h2_ref):
    h = h_ref[...]
    agg = agg_ref[...]
    u = lax.dot_general(
        h.astype(jnp.bfloat16), wh1_ref[...].astype(jnp.bfloat16),
        (((1,), (0,)), ((), ())), preferred_element_type=jnp.float32)
    u = u + lax.dot_general(
        agg.astype(jnp.bfloat16), wh2_ref[...].astype(jnp.bfloat16),
        (((1,), (0,)), ((), ())), preferred_element_type=jnp.float32)
    u = u + bh_ref[...]
    h2_ref[...] = h + u * jax.nn.sigmoid(u)


def _node_update(h, agg, Wh1, Wh2, bh):
    full = lambda i: (0, 0)
    return pl.pallas_call(
        _node_body,
        grid=(N_LIG_ // NODE_BLK,),
        in_specs=[
            pl.BlockSpec((NODE_BLK, HID), lambda i: (i, 0)),
            pl.BlockSpec((NODE_BLK, HID), lambda i: (i, 0)),
            pl.BlockSpec((HID, HID), full),
            pl.BlockSpec((HID, HID), full),
            pl.BlockSpec((1, HID), full),
        ],
        out_specs=pl.BlockSpec((NODE_BLK, HID), lambda i: (i, 0)),
        out_shape=jax.ShapeDtypeStruct((N_LIG_, HID), jnp.float32),
    )(h, agg, Wh1, Wh2, bh)


def kernel(protein_pos, protein_v, init_ligand_pos, W_prot, b_prot, W_lig,
           b_lig, W_e1, b_e1, W_e2, b_e2, W_h, b_h, W_x, b_x, W_v, b_v,
           batch_protein, init_ligand_v, batch_ligand, time_step, edge_index):
    ones = jnp.ones((N_PROT_, 1), jnp.float32)
    sums4 = jax.ops.segment_sum(jnp.concatenate([protein_pos, ones], axis=1),
                                batch_protein, num_segments=NUM_GRAPHS_)
    offset = sums4[:, :3] / jnp.maximum(sums4[:, 3:4], 1.0)
    oh_p = jax.nn.one_hot(batch_protein, NUM_GRAPHS_, dtype=jnp.float32)
    oh_l = jax.nn.one_hot(batch_ligand, NUM_GRAPHS_, dtype=jnp.float32)
    off_lig = oh_l @ offset
    p_pos = protein_pos - oh_p @ offset
    l_pos = init_ligand_pos - off_lig
    lig_onehot = jax.nn.one_hot(init_ligand_v, 13, dtype=jnp.float32)
    t_feat = oh_l @ (time_step.astype(jnp.float32) / NUM_TIMESTEPS_)[:, None]
    lig_feat = jnp.concatenate([lig_onehot, t_feat], axis=-1)
    h_lig = jnp.concatenate([lig_feat @ W_lig + b_lig,
                             jnp.ones((N_LIG_, 1), jnp.float32)], axis=-1)
    pos = jnp.concatenate([p_pos, l_pos], axis=0)
    W1d = W_e1[:HID]
    W1s = W_e1[HID:2 * HID]
    w_d2 = W_e1[2 * HID:2 * HID + 1]
    emb = HID - 1
    Td = jnp.concatenate(
        [protein_v @ (W_prot @ W1d[:emb]) + b_prot @ W1d[:emb],
         lig_feat @ (W_lig @ W1d[:emb]) + b_lig @ W1d[:emb] + W1d[emb]],
        axis=0)
    Ts = jnp.concatenate(
        [protein_v @ (W_prot @ W1s[:emb]) + b_prot @ W1s[:emb],
         lig_feat @ (W_lig @ W1s[:emb]) + b_lig @ W1s[:emb] + W1s[emb]],
        axis=0)
    src = edge_index[0]
    dst = edge_index[1]
    dst_remap = jnp.where((dst >= N_PROT_) & (dst < N_NODES_),
                          dst - N_PROT_,
                          jnp.int32(LIG_N)).astype(jnp.int32)
    posw = jnp.pad(pos, ((0, 0), (0, POSW - 3)))
    Wx8 = jnp.pad(W_x, ((0, 0), (0, 7)))
    bx8 = jnp.pad(b_x, (0, 7))[None, :]
    zeros_acc = jnp.zeros((S_CHUNK, HID), jnp.float32)
    zeros16 = jnp.zeros((S_CHUNK, POSW), jnp.float32)
    gath = []
    for s in range(K_SLICES):
        sl = slice(s * E_SLICE, (s + 1) * E_SLICE)
        gath.append((_sc_gather(Td, Ts, posw, dst[sl], src[sl]),
                     dst_remap[sl]))
    mids = []
    for (gd, gs, pd, ps), dsl in gath:
        m2, trans = _edge_mlp(gd, gs, pd, ps, w_d2, b_e1[None, :], W_e2,
                              b_e2[None, :], Wx8, bx8)
        mids.append((m2, trans, dsl))
    agg_p = []
    x_p = []
    for m2, trans, dsl in mids:
        parts, partsx = _sc_scatter(m2, trans, dsl, zeros_acc, zeros16)
        agg_p.append(parts)
        x_p.append(partsx)
    parts = sum(agg_p)
    partsx = sum(x_p)
    agg_l = (parts[0] + parts[1])[:LIG_N]
    x_l = (partsx[0] + partsx[1])[:LIG_N, :3]
    h2l = _node_update(h_lig, agg_l, W_h[:HID], W_h[HID:],
                       b_h[None, :])
    pred_ligand_pos = l_pos + x_l + off_lig
    pred_ligand_v = h2l @ W_v + b_v
    return pred_ligand_pos, pred_ligand_v
